# baseline (speedup 1.0000x reference)
import jax
import jax.numpy as jnp
from jax import lax
from jax.experimental import pallas as pl
from jax.experimental.pallas import tpu as pltpu
from jax.experimental.pallas import tpu_sc as plsc

_BUCKET_NUM = 100
_START_SMOOTH = 1
_D = 128
_N = 200000
_ROWS = 125
_ROWS_PAD = 128
_NCHUNKS = _N // _ROWS


def _tables_body(ep_ref, rm_ref, rv_ref, sm_ref, sv_ref, packed_ref):
    v1 = rv_ref[...]
    ok = v1 > 0.0
    v1s = jnp.where(ok, v1, 1.0)
    factor = jnp.clip(sv_ref[...] / v1s, 0.1, 10.0)
    s_raw = jnp.sqrt(factor)
    s = jnp.where(ok, s_raw, 1.0)
    b = jnp.where(ok, sm_ref[...] - rm_ref[...] * s_raw, 0.0)
    do = ep_ref[0, 0] >= _START_SMOOTH
    s = jnp.where(do, s, 1.0)
    b = jnp.where(do, b, 0.0)
    s_bits = lax.bitcast_convert_type(s.astype(jnp.bfloat16),
                                      jnp.uint16).astype(jnp.uint32)
    b_bits = lax.bitcast_convert_type(b.astype(jnp.bfloat16),
                                      jnp.uint16).astype(jnp.uint32)
    packed_ref[...] = lax.bitcast_convert_type((b_bits << 16) | s_bits,
                                               jnp.int32)


def _fold_tables(ep, rm, rv, sm, sv):
    return pl.pallas_call(
        _tables_body,
        out_shape=jax.ShapeDtypeStruct((_BUCKET_NUM, _D), jnp.int32),
        in_specs=[
            pl.BlockSpec(memory_space=pltpu.SMEM),
            pl.BlockSpec(memory_space=pltpu.VMEM),
            pl.BlockSpec(memory_space=pltpu.VMEM),
            pl.BlockSpec(memory_space=pltpu.VMEM),
            pl.BlockSpec(memory_space=pltpu.VMEM),
        ],
    )(ep, rm, rv, sm, sv)


def _sc_body(feat_hbm, bk_hbm, packed_hbm, out_hbm,
             packed_v, idx_a, idx_b, x_a, x_b, o_a, o_b,
             s_ia, s_ib, s_xa, s_xb, s_oa, s_ob):
    info = plsc.get_sparse_core_info()
    nc = info.num_cores
    wid = lax.axis_index("s") * nc + lax.axis_index("c")
    cpw = _NCHUNKS // (nc * info.num_subcores)
    g0 = wid * cpw

    pltpu.sync_copy(packed_hbm, packed_v)

    cols = [lax.iota(jnp.int32, 16) + 16 * j for j in range(_D // 16)]

    def start_in(g, idxv, xv, si, sx):
        pltpu.async_copy(bk_hbm.at[g], idxv, si)
        pltpu.async_copy(feat_hbm.at[pl.ds(g * _ROWS, _ROWS)],
                         xv.at[pl.ds(0, _ROWS)], sx)

    def wait_in(g, idxv, xv, si, sx):
        pltpu.make_async_copy(bk_hbm.at[g], idxv, si).wait()
        pltpu.make_async_copy(feat_hbm.at[pl.ds(g * _ROWS, _ROWS)],
                              xv.at[pl.ds(0, _ROWS)], sx).wait()

    def start_out(g, ov, so):
        pltpu.async_copy(ov.at[pl.ds(0, _ROWS)],
                         out_hbm.at[pl.ds(g * _ROWS, _ROWS)], so)

    def wait_out(g, ov, so):
        pltpu.make_async_copy(ov.at[pl.ds(0, _ROWS)],
                              out_hbm.at[pl.ds(g * _ROWS, _ROWS)], so).wait()

    def compute(idxv, xv, ov):
        @plsc.parallel_loop(0, _ROWS, step=1, unroll=5)
        def row(i):
            bb = plsc.load_gather(idxv, [jnp.broadcast_to(i, (16,))])
            for j in range(_D // 16):
                sl = pl.ds(j * 16, 16)
                w = plsc.load_gather(packed_v, [bb, cols[j]])
                sv = plsc.bitcast(w << 16, jnp.float32)
                bv = plsc.bitcast(w & jnp.int32(-65536), jnp.float32)
                ov[i, sl] = xv[i, sl] * sv + bv

    nt = cpw // 2
    start_in(g0, idx_a, x_a, s_ia, s_xa)

    def body(t, carry):
        ga = g0 + 2 * t
        gb = ga + 1
        start_in(gb, idx_b, x_b, s_ib, s_xb)
        wait_in(ga, idx_a, x_a, s_ia, s_xa)

        @pl.when(t > 0)
        def _():
            wait_out(ga - 2, o_a, s_oa)
        compute(idx_a, x_a, o_a)
        start_out(ga, o_a, s_oa)

        @pl.when(t < nt - 1)
        def _():
            start_in(ga + 2, idx_a, x_a, s_ia, s_xa)
        wait_in(gb, idx_b, x_b, s_ib, s_xb)

        @pl.when(t > 0)
        def _():
            wait_out(gb - 2, o_b, s_ob)
        compute(idx_b, x_b, o_b)
        start_out(gb, o_b, s_ob)
        return carry

    lax.fori_loop(0, nt, body, 0)
    wait_out(g0 + cpw - 2, o_a, s_oa)
    wait_out(g0 + cpw - 1, o_b, s_ob)


def kernel(features, buckets, epoch, running_mean_last_epoch,
           running_var_last_epoch, smoothed_mean_last_epoch,
           smoothed_var_last_epoch):
    ep = jnp.asarray(epoch, jnp.int32).reshape(1, 1)
    packed = _fold_tables(ep, running_mean_last_epoch,
                          running_var_last_epoch,
                          smoothed_mean_last_epoch,
                          smoothed_var_last_epoch)
    bk2d = jnp.pad(buckets.astype(jnp.int32).reshape(_NCHUNKS, _ROWS),
                   ((0, 0), (0, _ROWS_PAD - _ROWS)))
    mesh = plsc.VectorSubcoreMesh(core_axis_name="c", subcore_axis_name="s")
    sc = pl.kernel(
        _sc_body,
        out_type=jax.ShapeDtypeStruct((_N, _D), jnp.float32),
        mesh=mesh,
        scratch_types=[
            pltpu.VMEM((_BUCKET_NUM, _D), jnp.int32),
            pltpu.VMEM((_ROWS_PAD,), jnp.int32),
            pltpu.VMEM((_ROWS_PAD,), jnp.int32),
            pltpu.VMEM((_ROWS_PAD, _D), jnp.float32),
            pltpu.VMEM((_ROWS_PAD, _D), jnp.float32),
            pltpu.VMEM((_ROWS_PAD, _D), jnp.float32),
            pltpu.VMEM((_ROWS_PAD, _D), jnp.float32),
            pltpu.SemaphoreType.DMA,
            pltpu.SemaphoreType.DMA,
            pltpu.SemaphoreType.DMA,
            pltpu.SemaphoreType.DMA,
            pltpu.SemaphoreType.DMA,
            pltpu.SemaphoreType.DMA,
        ],
        compiler_params=pltpu.CompilerParams(use_tc_tiling_on_sc=False,
                                             needs_layout_passes=False),
    )
    return sc(features, bk2d, packed)


# speedup vs baseline: 1.0109x; 1.0109x over previous
import jax
import jax.numpy as jnp
from jax import lax
from jax.experimental import pallas as pl
from jax.experimental.pallas import tpu as pltpu
from jax.experimental.pallas import tpu_sc as plsc

_BUCKET_NUM = 100
_START_SMOOTH = 1
_D = 128
_N = 200000
_ROWS = 125
_ROWS_PAD = 128
_NCHUNKS = _N // _ROWS
_IDS_PAD = 6264


def _tables_body(ep_ref, rm_ref, rv_ref, sm_ref, sv_ref, packed_ref):
    v1 = rv_ref[...]
    ok = v1 > 0.0
    v1s = jnp.where(ok, v1, 1.0)
    factor = jnp.clip(sv_ref[...] / v1s, 0.1, 10.0)
    s_raw = jnp.sqrt(factor)
    s = jnp.where(ok, s_raw, 1.0)
    b = jnp.where(ok, sm_ref[...] - rm_ref[...] * s_raw, 0.0)
    do = ep_ref[0, 0] >= _START_SMOOTH
    s = jnp.where(do, s, 1.0)
    b = jnp.where(do, b, 0.0)
    s_bits = lax.bitcast_convert_type(s.astype(jnp.bfloat16),
                                      jnp.uint16).astype(jnp.uint32)
    b_bits = lax.bitcast_convert_type(b.astype(jnp.bfloat16),
                                      jnp.uint16).astype(jnp.uint32)
    packed_ref[...] = lax.bitcast_convert_type((b_bits << 16) | s_bits,
                                               jnp.int32)


def _fold_tables(ep, rm, rv, sm, sv):
    return pl.pallas_call(
        _tables_body,
        out_shape=jax.ShapeDtypeStruct((_BUCKET_NUM, _D), jnp.int32),
        in_specs=[
            pl.BlockSpec(memory_space=pltpu.SMEM),
            pl.BlockSpec(memory_space=pltpu.VMEM),
            pl.BlockSpec(memory_space=pltpu.VMEM),
            pl.BlockSpec(memory_space=pltpu.VMEM),
            pl.BlockSpec(memory_space=pltpu.VMEM),
        ],
    )(ep, rm, rv, sm, sv)


def _sc_body(feat_hbm, bk_hbm, packed_hbm, out_hbm,
             packed_v, idx_v, x_a, x_b, o_a, o_b,
             s_xa, s_xb, s_oa, s_ob):
    info = plsc.get_sparse_core_info()
    nc = info.num_cores
    wid = lax.axis_index("s") * nc + lax.axis_index("c")
    cpw = _NCHUNKS // (nc * info.num_subcores)
    g0 = wid * cpw

    start = g0 * _ROWS
    base = (start // 8) * 8
    delta = start - base
    pltpu.async_copy(bk_hbm.at[pl.ds(base, _IDS_PAD)],
                     idx_v.at[pl.ds(0, _IDS_PAD)], s_xa)
    pltpu.sync_copy(packed_hbm, packed_v)
    pltpu.make_async_copy(bk_hbm.at[pl.ds(base, _IDS_PAD)],
                          idx_v.at[pl.ds(0, _IDS_PAD)], s_xa).wait()

    cols = [lax.iota(jnp.int32, 16) + 16 * j for j in range(_D // 16)]

    def start_in(g, xv, sx):
        pltpu.async_copy(feat_hbm.at[pl.ds(g * _ROWS, _ROWS)],
                         xv.at[pl.ds(0, _ROWS)], sx)

    def wait_in(g, xv, sx):
        pltpu.make_async_copy(feat_hbm.at[pl.ds(g * _ROWS, _ROWS)],
                              xv.at[pl.ds(0, _ROWS)], sx).wait()

    def start_out(g, ov, so):
        pltpu.async_copy(ov.at[pl.ds(0, _ROWS)],
                         out_hbm.at[pl.ds(g * _ROWS, _ROWS)], so)

    def wait_out(g, ov, so):
        pltpu.make_async_copy(ov.at[pl.ds(0, _ROWS)],
                              out_hbm.at[pl.ds(g * _ROWS, _ROWS)], so).wait()

    def compute(off, xv, ov):
        @plsc.parallel_loop(0, _ROWS, step=1, unroll=5)
        def row(i):
            bb = plsc.load_gather(idx_v,
                                  [jnp.broadcast_to(delta + off + i, (16,))])
            for j in range(_D // 16):
                sl = pl.ds(j * 16, 16)
                w = plsc.load_gather(packed_v, [bb, cols[j]])
                sv = plsc.bitcast(w << 16, jnp.float32)
                bv = plsc.bitcast(w & jnp.int32(-65536), jnp.float32)
                ov[i, sl] = xv[i, sl] * sv + bv

    nt = cpw // 2
    start_in(g0, x_a, s_xa)

    def body(t, carry):
        ga = g0 + 2 * t
        gb = ga + 1
        start_in(gb, x_b, s_xb)
        wait_in(ga, x_a, s_xa)

        @pl.when(t > 0)
        def _():
            wait_out(ga - 2, o_a, s_oa)
        compute(2 * t * _ROWS, x_a, o_a)
        start_out(ga, o_a, s_oa)

        @pl.when(t < nt - 1)
        def _():
            start_in(ga + 2, x_a, s_xa)
        wait_in(gb, x_b, s_xb)

        @pl.when(t > 0)
        def _():
            wait_out(gb - 2, o_b, s_ob)
        compute((2 * t + 1) * _ROWS, x_b, o_b)
        start_out(gb, o_b, s_ob)
        return carry

    lax.fori_loop(0, nt, body, 0)
    wait_out(g0 + cpw - 2, o_a, s_oa)
    wait_out(g0 + cpw - 1, o_b, s_ob)


def kernel(features, buckets, epoch, running_mean_last_epoch,
           running_var_last_epoch, smoothed_mean_last_epoch,
           smoothed_var_last_epoch):
    ep = jnp.asarray(epoch, jnp.int32).reshape(1, 1)
    packed = _fold_tables(ep, running_mean_last_epoch,
                          running_var_last_epoch,
                          smoothed_mean_last_epoch,
                          smoothed_var_last_epoch)
    bk1d = jnp.pad(buckets.astype(jnp.int32), (0, _IDS_PAD - _N % _IDS_PAD))
    mesh = plsc.VectorSubcoreMesh(core_axis_name="c", subcore_axis_name="s")
    sc = pl.kernel(
        _sc_body,
        out_type=jax.ShapeDtypeStruct((_N, _D), jnp.float32),
        mesh=mesh,
        scratch_types=[
            pltpu.VMEM((_BUCKET_NUM, _D), jnp.int32),
            pltpu.VMEM((_IDS_PAD,), jnp.int32),
            pltpu.VMEM((_ROWS_PAD, _D), jnp.float32),
            pltpu.VMEM((_ROWS_PAD, _D), jnp.float32),
            pltpu.VMEM((_ROWS_PAD, _D), jnp.float32),
            pltpu.VMEM((_ROWS_PAD, _D), jnp.float32),
            pltpu.SemaphoreType.DMA,
            pltpu.SemaphoreType.DMA,
            pltpu.SemaphoreType.DMA,
            pltpu.SemaphoreType.DMA,
        ],
        compiler_params=pltpu.CompilerParams(use_tc_tiling_on_sc=False,
                                             needs_layout_passes=False),
    )
    return sc(features, bk1d, packed)
